# baseline (speedup 1.0000x reference)
import functools

import jax
import jax.numpy as jnp
from jax import lax
from jax.experimental import pallas as pl
from jax.experimental.pallas import tpu as pltpu
from jax.experimental.pallas import tpu_sc as plsc

N = 10000
E = 320000
D = 128
H = 16
B = 64
NC = 2
NS = 16
NW = NC * NS
EB = 80
NB = 125
RPT = N // NS
N_SP = 10240
ZPT = N_SP // NS
FPR = N_SP * H // 128
DEG_T = 5
DEG_CH = N // DEG_T
NBUF = 25
K_G = 13
L_S = NBUF - K_G

_mesh = plsc.VectorSubcoreMesh(
    core_axis_name="c", subcore_axis_name="s", num_cores=NC, num_subcores=NS)
_sc_params = pltpu.CompilerParams(use_tc_tiling_on_sc=False)


def _sc_hist_body(dst_hbm, deg_out, dst_v, ones_v, zed_v, deg_sp, ssem):
    c = lax.axis_index("c")
    s = lax.axis_index("s")
    wid = c * NS + s

    ld = pltpu.async_copy(dst_hbm.at[wid], dst_v, ssem)

    @pl.loop(0, EB // 16)
    def _fill(i):
        ones_v[pl.ds(i * 16, 16)] = jnp.ones((16,), jnp.float32)

    @pl.when(s < DEG_T)
    def _zero():
        @pl.loop(0, DEG_CH // 16)
        def _z(i):
            zed_v[pl.ds(i * 16, 16)] = jnp.zeros((16,), jnp.float32)
        pltpu.sync_copy(zed_v, deg_sp.at[pl.ds(s * DEG_CH, DEG_CH)])

    ld.wait()
    plsc.subcore_barrier()

    @pl.loop(0, NB)
    def _fire(j):
        pltpu.async_copy(ones_v, deg_sp.at[dst_v.at[j]], ssem, add=True)

    @pl.loop(0, NB)
    def _drain(j):
        pltpu.make_async_copy(ones_v, deg_sp.at[dst_v.at[j]], ssem).wait()

    plsc.subcore_barrier()

    @pl.when(s < DEG_T)
    def _export():
        pltpu.sync_copy(deg_sp.at[pl.ds(s * DEG_CH, DEG_CH)],
                        deg_out.at[c, pl.ds(s * DEG_CH, DEG_CH)])


_sc_hist = pl.kernel(
    _sc_hist_body,
    out_type=jax.ShapeDtypeStruct((NC, N), jnp.float32),
    mesh=_mesh,
    scratch_types=[
        pltpu.VMEM((NB, EB), jnp.int32),
        pltpu.VMEM((EB,), jnp.float32),
        pltpu.VMEM((DEG_CH,), jnp.float32),
        pltpu.VMEM_SHARED((N,), jnp.float32),
        pltpu.SemaphoreType.DMA,
    ],
    compiler_params=_sc_params,
)


def _sc_mp_body(xws_hbm, dinv_hbm, src_hbm, dst_hbm, out_hbm,
                src_v, dst_v, rows_v, zrow_v, iota_v, acc_v, dinv_v,
                xws_sp, out_sp, gsem, ssem):
    c = lax.axis_index("c")
    s = lax.axis_index("s")
    wid = c * NS + s

    d1 = pltpu.async_copy(src_hbm.at[wid], src_v, gsem)
    d2 = pltpu.async_copy(dst_hbm.at[wid], dst_v, gsem)
    d3 = pltpu.async_copy(xws_hbm.at[pl.ds(s * RPT, RPT)],
                          xws_sp.at[pl.ds(s * RPT, RPT)], gsem)

    @pl.loop(0, ZPT)
    def _z(i):
        zrow_v[i, :] = jnp.zeros((16,), jnp.float32)

    d4 = pltpu.async_copy(zrow_v, out_sp.at[pl.ds(s * ZPT, ZPT)], gsem)

    @pl.when(s == 0)
    def _zpad():
        pltpu.sync_copy(zrow_v.at[pl.ds(0, N_SP - N)],
                        xws_sp.at[pl.ds(N, N_SP - N)])

    d1.wait()
    d2.wait()
    d3.wait()
    d4.wait()
    plsc.subcore_barrier()

    def _gather(j, b):
        return pltpu.async_copy(xws_sp.at[src_v.at[j]], rows_v.at[b], gsem)

    def _scatter(j, b):
        return pltpu.async_copy(rows_v.at[b], out_sp.at[dst_v.at[j]], ssem,
                                add=True)

    def _wait_gather(j, b):
        pltpu.make_async_copy(xws_sp.at[src_v.at[j]], rows_v.at[b],
                              gsem).wait()

    def _wait_scatter(j, b):
        pltpu.make_async_copy(rows_v.at[b], out_sp.at[dst_v.at[j]],
                              ssem).wait()

    for jj in range(K_G):
        _gather(jj, jj)

    @pl.loop(0, NB // NBUF)
    def _outer(g):
        for b in range(NBUF):
            j = g * NBUF + b
            _wait_gather(j, b)
            _scatter(j, b)

            @pl.when(j >= L_S)
            def _():
                _wait_scatter(j - L_S, (b - L_S) % NBUF)

            @pl.when(j + K_G < NB)
            def _():
                _gather(j + K_G, (b + K_G) % NBUF)

    for t in range(L_S):
        j = NB - L_S + t
        _wait_scatter(j, j % NBUF)

    @pl.when(c == 0)
    def _selfloop():
        @pl.loop(0, ZPT // 128)
        def _qi(q):
            @pl.loop(0, 8)
            def _ii(i):
                iota_v[q, pl.ds(i * 16, 16)] = (
                    lax.iota(jnp.int32, 16) + s * ZPT + q * 128 + i * 16)
        pltpu.sync_copy(xws_sp.at[pl.ds(s * ZPT, ZPT)], zrow_v)

        @pl.loop(0, ZPT // 128)
        def _qs(q):
            pltpu.sync_copy(zrow_v.at[pl.ds(q * 128, 128)],
                            out_sp.at[iota_v.at[q]], add=True)

    plsc.subcore_barrier()

    pltpu.sync_copy(out_sp.at[pl.ds(s * ZPT, ZPT)], acc_v)
    pltpu.sync_copy(dinv_hbm.at[pl.ds(s * ZPT, ZPT)], dinv_v)

    @pl.loop(0, ZPT // 16)
    def _scale(t):
        dvec = dinv_v[pl.ds(t * 16, 16)]
        for k in range(16):
            i = t * 16 + k
            acc_v[i, :] = acc_v[i, :] * dvec[k]

    pltpu.sync_copy(acc_v, out_hbm.at[c, pl.ds(s * ZPT, ZPT)])


_sc_mp = pl.kernel(
    _sc_mp_body,
    out_type=jax.ShapeDtypeStruct((NC, N_SP, H), jnp.float32),
    mesh=_mesh,
    scratch_types=[
        pltpu.VMEM((NB, EB), jnp.int32),
        pltpu.VMEM((NB, EB), jnp.int32),
        pltpu.VMEM((NBUF, EB, H), jnp.float32),
        pltpu.VMEM((ZPT, H), jnp.float32),
        pltpu.VMEM((ZPT // 128, 128), jnp.int32),
        pltpu.VMEM((ZPT, H), jnp.float32),
        pltpu.VMEM((ZPT,), jnp.float32),
        pltpu.VMEM_SHARED((N_SP, H), jnp.float32),
        pltpu.VMEM_SHARED((N_SP, H), jnp.float32),
        pltpu.SemaphoreType.DMA,
        pltpu.SemaphoreType.DMA,
    ],
    compiler_params=_sc_params,
)


def _tc1a_body(x_ref, w_ref, xw_ref):
    xw_ref[...] = jnp.dot(x_ref[...], w_ref[...],
                          preferred_element_type=jnp.float32)


_tc1a = pl.pallas_call(
    _tc1a_body,
    out_shape=jax.ShapeDtypeStruct((N, H), jnp.float32),
)


def _tc1b_body(xw_ref, dp_ref, bt_ref, xws_ref, dinv_ref, bt_out_ref):
    deg = dp_ref[...].sum(axis=0, keepdims=True) + 1.0
    dinv = lax.rsqrt(deg).reshape(N, 1)
    dinv_ref[0:N] = dinv.reshape(N)
    dinv_ref[N:N_SP] = jnp.zeros((N_SP - N,), jnp.float32)
    xws_ref[...] = xw_ref[...] * dinv
    bt_out_ref[...] = bt_ref[...]


_tc1b = pl.pallas_call(
    _tc1b_body,
    out_shape=[
        jax.ShapeDtypeStruct((N, H), jnp.float32),
        jax.ShapeDtypeStruct((N_SP,), jnp.float32),
        jax.ShapeDtypeStruct((8, FPR), jnp.int32),
    ],
)


def _tc2_body(p0_ref, p1_ref, batcht_ref,
              mri_ref, cog_ref, clin_ref, gen_ref,
              b1f_ref, mw_ref, mb_ref, cw_ref, cb_ref, lw_ref, lb_ref,
              gw_ref, gb_ref, w1a_ref, w1b_ref, w1c_ref, w1d_ref, w1e_ref,
              cb1_ref, w2_ref, cb2_ref, out_ref):
    h = jnp.maximum(p0_ref[...] + p1_ref[...] + b1f_ref[...], 0.0)

    iota_b = lax.broadcasted_iota(jnp.int32, (B, FPR), 0)
    sums = jnp.zeros((B, H), jnp.float32)
    cnt = jnp.zeros((B, 1), jnp.float32)
    for k in range(8):
        ok = jnp.where(batcht_ref[k:k + 1, :] == iota_b, 1.0, 0.0)
        sums = sums + jnp.dot(ok, h[:, 16 * k:16 * (k + 1)],
                              preferred_element_type=jnp.float32)
        cnt = cnt + ok.sum(axis=1, keepdims=True)
    ge = sums / jnp.maximum(cnt, 1.0)

    relu = lambda v: jnp.maximum(v, 0.0)
    dot = functools.partial(jnp.dot, preferred_element_type=jnp.float32)
    mri = relu(dot(mri_ref[...], mw_ref[...]) + mb_ref[...])
    cog = relu(dot(cog_ref[...], cw_ref[...]) + cb_ref[...])
    clin = relu(dot(clin_ref[...], lw_ref[...]) + lb_ref[...])
    gen = relu(dot(gen_ref[...], gw_ref[...]) + gb_ref[...])

    h2 = relu(dot(ge, w1a_ref[...]) + dot(mri, w1b_ref[...])
              + dot(cog, w1c_ref[...]) + dot(clin, w1d_ref[...])
              + dot(gen, w1e_ref[...]) + cb1_ref[...])
    logits = dot(h2, w2_ref[...]) + cb2_ref[...]
    m = jnp.max(logits, axis=1, keepdims=True)
    lse = m + jnp.log(jnp.sum(jnp.exp(logits - m), axis=1, keepdims=True))
    out_ref[...] = logits - lse


_tc2 = pl.pallas_call(
    _tc2_body,
    out_shape=jax.ShapeDtypeStruct((B, 3), jnp.float32),
)


def kernel(x, edge_index, batch, mri_features, cog_features, clin_features,
           genetic_features, W1, b1, mri_W, mri_b, cog_W, cog_b, clin_W,
           clin_b, gen_W, gen_b, cW1, cb1, cW2, cb2):
    src2 = edge_index[0].reshape(NW, NB, EB)
    dst2 = edge_index[1].reshape(NW, NB, EB)

    batcht0 = jnp.pad(batch.reshape(N // 8, 8), ((0, (N_SP - N) // 8), (0, 0)),
                      constant_values=B).T
    b1f = jnp.tile(b1.reshape(1, H), (1, 8))

    xw = _tc1a(x, W1)
    deg_parts = _sc_hist(dst2)
    xws, dinv, batcht = _tc1b(xw, deg_parts, batcht0)
    out_flat = _sc_mp(xws, dinv, src2, dst2).reshape(NC, FPR, 128)

    return _tc2(
        out_flat[0], out_flat[1], batcht,
        mri_features, cog_features, clin_features, genetic_features,
        b1f, mri_W, mri_b.reshape(1, 4), cog_W,
        cog_b.reshape(1, 4), clin_W, clin_b.reshape(1, 4), gen_W,
        gen_b.reshape(1, 4), cW1[0:16], cW1[16:20], cW1[20:24], cW1[24:28],
        cW1[28:32], cb1.reshape(1, 16), cW2, cb2.reshape(1, 3))


# speedup vs baseline: 1.1359x; 1.1359x over previous
import functools

import jax
import jax.numpy as jnp
from jax import lax
from jax.experimental import pallas as pl
from jax.experimental.pallas import tpu as pltpu
from jax.experimental.pallas import tpu_sc as plsc

N = 10000
E = 320000
D = 128
H = 16
B = 64
NC = 2
NS = 16
NW = NC * NS
EB = 80
NB = 125
RPT = N // NS
N_SP = 10240
ZPT = N_SP // NS
FPR = N_SP * H // 128
DEG_T = 5
DEG_CH = N // DEG_T
NBUF = 25
K_G = 13
L_S = NBUF - K_G

_mesh = plsc.VectorSubcoreMesh(
    core_axis_name="c", subcore_axis_name="s", num_cores=NC, num_subcores=NS)
_sc_params = pltpu.CompilerParams(use_tc_tiling_on_sc=False)


def _sc_hist_body(ei_hbm, deg_out, dst_v, ones_v, zed_v, deg_sp, ssem):
    c = lax.axis_index("c")
    s = lax.axis_index("s")
    wid = c * NS + s

    ld = pltpu.async_copy(ei_hbm.at[1, wid], dst_v, ssem)

    @pl.loop(0, EB // 16)
    def _fill(i):
        ones_v[pl.ds(i * 16, 16)] = jnp.ones((16,), jnp.float32)

    @pl.when(s < DEG_T)
    def _zero():
        @pl.loop(0, DEG_CH // 16)
        def _z(i):
            zed_v[pl.ds(i * 16, 16)] = jnp.zeros((16,), jnp.float32)
        pltpu.sync_copy(zed_v, deg_sp.at[pl.ds(s * DEG_CH, DEG_CH)])

    ld.wait()
    plsc.subcore_barrier()

    @pl.loop(0, NB)
    def _fire(j):
        pltpu.async_copy(ones_v, deg_sp.at[dst_v.at[j]], ssem, add=True)

    @pl.loop(0, NB)
    def _drain(j):
        pltpu.make_async_copy(ones_v, deg_sp.at[dst_v.at[j]], ssem).wait()

    plsc.subcore_barrier()

    @pl.when(s < DEG_T)
    def _export():
        pltpu.sync_copy(deg_sp.at[pl.ds(s * DEG_CH, DEG_CH)],
                        deg_out.at[c, pl.ds(s * DEG_CH, DEG_CH)])


_sc_hist = pl.kernel(
    _sc_hist_body,
    out_type=jax.ShapeDtypeStruct((NC, N), jnp.float32),
    mesh=_mesh,
    scratch_types=[
        pltpu.VMEM((NB, EB), jnp.int32),
        pltpu.VMEM((EB,), jnp.float32),
        pltpu.VMEM((DEG_CH,), jnp.float32),
        pltpu.VMEM_SHARED((N,), jnp.float32),
        pltpu.SemaphoreType.DMA,
    ],
    compiler_params=_sc_params,
)


def _sc_mp_body(xws_hbm, dinv_hbm, ei_hbm, out_hbm,
                src_v, dst_v, rows_v, zrow_v, iota_v, acc_v, dinv_v,
                xws_sp, out_sp, gsem, ssem):
    c = lax.axis_index("c")
    s = lax.axis_index("s")
    wid = c * NS + s

    d1 = pltpu.async_copy(ei_hbm.at[0, wid], src_v, gsem)
    d2 = pltpu.async_copy(ei_hbm.at[1, wid], dst_v, gsem)
    d3 = pltpu.async_copy(xws_hbm.at[pl.ds(s * RPT, RPT)],
                          xws_sp.at[pl.ds(s * RPT, RPT)], gsem)

    @pl.loop(0, ZPT)
    def _z(i):
        zrow_v[i, :] = jnp.zeros((16,), jnp.float32)

    d4 = pltpu.async_copy(zrow_v, out_sp.at[pl.ds(s * ZPT, ZPT)], gsem)

    @pl.when(s == 0)
    def _zpad():
        pltpu.sync_copy(zrow_v.at[pl.ds(0, N_SP - N)],
                        xws_sp.at[pl.ds(N, N_SP - N)])

    d1.wait()
    d2.wait()
    d3.wait()
    d4.wait()
    plsc.subcore_barrier()

    def _gather(j, b):
        return pltpu.async_copy(xws_sp.at[src_v.at[j]], rows_v.at[b], gsem)

    def _scatter(j, b):
        return pltpu.async_copy(rows_v.at[b], out_sp.at[dst_v.at[j]], ssem,
                                add=True)

    def _wait_gather(j, b):
        pltpu.make_async_copy(xws_sp.at[src_v.at[j]], rows_v.at[b],
                              gsem).wait()

    def _wait_scatter(j, b):
        pltpu.make_async_copy(rows_v.at[b], out_sp.at[dst_v.at[j]],
                              ssem).wait()

    for jj in range(K_G):
        _gather(jj, jj)

    @pl.loop(0, NB // NBUF)
    def _outer(g):
        for b in range(NBUF):
            j = g * NBUF + b
            _wait_gather(j, b)
            _scatter(j, b)

            @pl.when(j >= L_S)
            def _():
                _wait_scatter(j - L_S, (b - L_S) % NBUF)

            @pl.when(j + K_G < NB)
            def _():
                _gather(j + K_G, (b + K_G) % NBUF)

    for t in range(L_S):
        j = NB - L_S + t
        _wait_scatter(j, j % NBUF)

    @pl.when(c == 0)
    def _selfloop():
        @pl.loop(0, ZPT // 128)
        def _qi(q):
            @pl.loop(0, 8)
            def _ii(i):
                iota_v[q, pl.ds(i * 16, 16)] = (
                    lax.iota(jnp.int32, 16) + s * ZPT + q * 128 + i * 16)
        pltpu.sync_copy(xws_sp.at[pl.ds(s * ZPT, ZPT)], zrow_v)

        @pl.loop(0, ZPT // 128)
        def _qs(q):
            pltpu.sync_copy(zrow_v.at[pl.ds(q * 128, 128)],
                            out_sp.at[iota_v.at[q]], add=True)

    plsc.subcore_barrier()

    pltpu.sync_copy(out_sp.at[pl.ds(s * ZPT, ZPT)], acc_v)
    pltpu.sync_copy(dinv_hbm.at[pl.ds(s * ZPT, ZPT)], dinv_v)

    @pl.loop(0, ZPT // 16)
    def _scale(t):
        dvec = dinv_v[pl.ds(t * 16, 16)]
        for k in range(16):
            i = t * 16 + k
            acc_v[i, :] = acc_v[i, :] * dvec[k]

    pltpu.sync_copy(acc_v, out_hbm.at[c, pl.ds(s * ZPT, ZPT)])


_sc_mp = pl.kernel(
    _sc_mp_body,
    out_type=jax.ShapeDtypeStruct((NC, N_SP, H), jnp.float32),
    mesh=_mesh,
    scratch_types=[
        pltpu.VMEM((NB, EB), jnp.int32),
        pltpu.VMEM((NB, EB), jnp.int32),
        pltpu.VMEM((NBUF, EB, H), jnp.float32),
        pltpu.VMEM((ZPT, H), jnp.float32),
        pltpu.VMEM((ZPT // 128, 128), jnp.int32),
        pltpu.VMEM((ZPT, H), jnp.float32),
        pltpu.VMEM((ZPT,), jnp.float32),
        pltpu.VMEM_SHARED((N_SP, H), jnp.float32),
        pltpu.VMEM_SHARED((N_SP, H), jnp.float32),
        pltpu.SemaphoreType.DMA,
        pltpu.SemaphoreType.DMA,
    ],
    compiler_params=_sc_params,
)


def _tc1a_body(x_ref, w_ref, xw_ref):
    xw_ref[...] = jnp.dot(x_ref[...], w_ref[...],
                          preferred_element_type=jnp.float32)


_tc1a = pl.pallas_call(
    _tc1a_body,
    out_shape=jax.ShapeDtypeStruct((N, H), jnp.float32),
)


def _tc1b_body(xw_ref, dp_ref, xws_ref, dinv_ref):
    deg = dp_ref[...].sum(axis=0, keepdims=True) + 1.0
    dinv = lax.rsqrt(deg).reshape(N, 1)
    dinv_ref[0:N] = dinv.reshape(N)
    dinv_ref[N:N_SP] = jnp.zeros((N_SP - N,), jnp.float32)
    xws_ref[...] = xw_ref[...] * dinv


_tc1b = pl.pallas_call(
    _tc1b_body,
    out_shape=[
        jax.ShapeDtypeStruct((N, H), jnp.float32),
        jax.ShapeDtypeStruct((N_SP,), jnp.float32),
    ],
)


def _tc2_body(p0_ref, p1_ref, batcht_ref,
              mri_ref, cog_ref, clin_ref, gen_ref,
              b1f_ref, mw_ref, mb_ref, cw_ref, cb_ref, lw_ref, lb_ref,
              gw_ref, gb_ref, w1a_ref, w1b_ref, w1c_ref, w1d_ref, w1e_ref,
              cb1_ref, w2_ref, cb2_ref, out_ref):
    h = jnp.maximum(p0_ref[...] + p1_ref[...] + b1f_ref[...], 0.0)

    iota_b = lax.broadcasted_iota(jnp.int32, (B, FPR), 0)
    sums = jnp.zeros((B, H), jnp.float32)
    cnt = jnp.zeros((B, 1), jnp.float32)
    for k in range(8):
        ok = jnp.where(batcht_ref[k:k + 1, :] == iota_b, 1.0, 0.0)
        sums = sums + jnp.dot(ok, h[:, 16 * k:16 * (k + 1)],
                              preferred_element_type=jnp.float32)
        cnt = cnt + ok.sum(axis=1, keepdims=True)
    ge = sums / jnp.maximum(cnt, 1.0)

    relu = lambda v: jnp.maximum(v, 0.0)
    dot = functools.partial(jnp.dot, preferred_element_type=jnp.float32)
    mri = relu(dot(mri_ref[...], mw_ref[...]) + mb_ref[...])
    cog = relu(dot(cog_ref[...], cw_ref[...]) + cb_ref[...])
    clin = relu(dot(clin_ref[...], lw_ref[...]) + lb_ref[...])
    gen = relu(dot(gen_ref[...], gw_ref[...]) + gb_ref[...])

    h2 = relu(dot(ge, w1a_ref[...]) + dot(mri, w1b_ref[...])
              + dot(cog, w1c_ref[...]) + dot(clin, w1d_ref[...])
              + dot(gen, w1e_ref[...]) + cb1_ref[...])
    logits = dot(h2, w2_ref[...]) + cb2_ref[...]
    m = jnp.max(logits, axis=1, keepdims=True)
    lse = m + jnp.log(jnp.sum(jnp.exp(logits - m), axis=1, keepdims=True))
    out_ref[...] = logits - lse


_tc2 = pl.pallas_call(
    _tc2_body,
    out_shape=jax.ShapeDtypeStruct((B, 3), jnp.float32),
)


def kernel(x, edge_index, batch, mri_features, cog_features, clin_features,
           genetic_features, W1, b1, mri_W, mri_b, cog_W, cog_b, clin_W,
           clin_b, gen_W, gen_b, cW1, cb1, cW2, cb2):
    ei4 = edge_index.reshape(2, NW, NB, EB)

    batcht = jnp.pad(batch.reshape(N // 8, 8), ((0, (N_SP - N) // 8), (0, 0)),
                     constant_values=B).T
    b1f = jnp.tile(b1.reshape(1, H), (1, 8))

    xw = _tc1a(x, W1)
    deg_parts = _sc_hist(ei4)
    xws, dinv = _tc1b(xw, deg_parts)
    out_flat = _sc_mp(xws, dinv, ei4).reshape(NC, FPR, 128)

    return _tc2(
        out_flat[0], out_flat[1], batcht,
        mri_features, cog_features, clin_features, genetic_features,
        b1f, mri_W, mri_b.reshape(1, 4), cog_W,
        cog_b.reshape(1, 4), clin_W, clin_b.reshape(1, 4), gen_W,
        gen_b.reshape(1, 4), cW1[0:16], cW1[16:20], cW1[20:24], cW1[24:28],
        cW1[28:32], cb1.reshape(1, 16), cW2, cb2.reshape(1, 3))
